# baseline (speedup 1.0000x reference)
import functools

import jax
import jax.numpy as jnp
from jax import lax
from jax.experimental import pallas as pl
from jax.experimental.pallas import tpu as pltpu
from jax.experimental.pallas import tpu_sc as plsc

_N = 500000
_C = 100
_NBINS = 10

_NSC = 128000
_NTC = _N - _NSC
_NW = 32
_RPW = _NSC // _NW
_CHUNK = 400
_NCH = _RPW // _CHUNK
_GRP = _CHUNK // 16

_BLK = 12000
_NBLKA = _NTC // _BLK
_BN_ROWS = 5000
_BN_COLS = 100
_BN_BLK = 1000


def _rows_kernel(x_ref, tgt_ref, conf_ref, acc_ref):
    xt = jnp.swapaxes(x_ref[...], 0, 1)
    m = jnp.max(xt, axis=0, keepdims=True)
    e = jnp.exp(xt - m)
    s = jnp.sum(e, axis=0, keepdims=True)
    conf_ref[0] = 1.0 / s
    iota = jax.lax.broadcasted_iota(jnp.int32, xt.shape, 0)
    tval = jnp.max(jnp.where(iota == tgt_ref[0], xt, -jnp.inf), axis=0, keepdims=True)
    acc_ref[0] = (tval == m).astype(jnp.float32)


def _sc_rows(x_hbm, tgt_hbm, conf_hbm, acc_hbm, xbuf, tbuf, cbuf, abuf):
    wid = lax.axis_index("s") * 2 + lax.axis_index("c")
    riota = lax.iota(jnp.int32, 16)
    m_a6 = riota < 4
    m_b6 = riota >= 4
    m_b12 = riota < 8
    lane_eq = [riota == j for j in range(16)]
    pos = [16 * k + riota for k in range(13)]
    neg = jnp.float32(-jnp.inf)
    zero16 = jnp.zeros((16,), jnp.float32)

    rot = [jnp.bitwise_and(riota + sh, 15) for sh in (1, 2, 4, 8)]

    def allmax(v):
        for r in rot:
            v = jnp.maximum(v, jnp.take(v, r))
        return v

    def allsum(v):
        for r in rot:
            v = v + jnp.take(v, r)
        return v

    def row_stats(v, masks, krange, poff, tgt_splat):
        ts = tgt_splat + poff
        am = neg * jnp.ones((16,), jnp.float32)
        for k in krange:
            vk = jnp.where(masks[k], v[k], neg) if k in masks else v[k]
            am = jnp.maximum(am, vk)
        m = allmax(am)
        ae = zero16
        tv = neg * jnp.ones((16,), jnp.float32)
        for k in krange:
            e = jnp.exp(v[k] - m)
            if k in masks:
                e = jnp.where(masks[k], e, 0.0)
            ae = ae + e
            tv = jnp.maximum(tv, jnp.where(pos[k] == ts, v[k], neg))
        return m, allsum(ae), allmax(tv)

    def chunk_body(ch, carry):
        obase = wid * _RPW + ch * _CHUNK
        rbase = _NTC + obase
        pltpu.sync_copy(x_hbm.at[pl.ds(rbase * _C, _CHUNK * _C)], xbuf.at[pl.ds(0, _CHUNK * _C)])
        pltpu.sync_copy(tgt_hbm.at[pl.ds(rbase, _CHUNK)], tbuf)

        def group_body(g, carry2):
            tvec = tbuf[pl.ds(g * 16, 16)]
            s_vec = jnp.ones((16,), jnp.float32)
            m_vec = zero16
            t_vec = zero16
            for pr in range(8):
                base = g * 16 * _C + pr * 2 * _C
                v = [xbuf[pl.ds(base + 16 * k, 16)] for k in range(13)]
                ta = jnp.take(tvec, jnp.bitwise_and(riota * 0 + 2 * pr, 15))
                tb = jnp.take(tvec, jnp.bitwise_and(riota * 0 + 2 * pr + 1, 15))
                ma, sa, tva = row_stats(v, {6: m_a6}, range(7), 0, ta)
                mb, sb, tvb = row_stats(v, {6: m_b6, 12: m_b12}, range(6, 13), _C, tb)
                for val, vecidx in ((ma, 0), (mb, 1)):
                    m_vec = jnp.where(lane_eq[2 * pr + vecidx], val, m_vec)
                for val, vecidx in ((sa, 0), (sb, 1)):
                    s_vec = jnp.where(lane_eq[2 * pr + vecidx], val, s_vec)
                for val, vecidx in ((tva, 0), (tvb, 1)):
                    t_vec = jnp.where(lane_eq[2 * pr + vecidx], val, t_vec)
            cbuf[pl.ds(g * 16, 16)] = 1.0 / s_vec
            abuf[pl.ds(g * 16, 16)] = jnp.where(t_vec == m_vec, 1.0, 0.0)
            return carry2

        lax.fori_loop(0, _CHUNK // 16, group_body, 0)
        pltpu.sync_copy(cbuf, conf_hbm.at[pl.ds(obase, _CHUNK)])
        pltpu.sync_copy(abuf, acc_hbm.at[pl.ds(obase, _CHUNK)])
        return carry

    lax.fori_loop(0, _NCH, chunk_body, 0)


def _hist_kernel(bounds_ref, conf_ref, accv_ref, out_ref, cnt_ref, sum_ref, *, nsteps):
    step = pl.program_id(0)

    @pl.when(step == 0)
    def _init():
        cnt_ref[...] = jnp.zeros_like(cnt_ref)
        sum_ref[...] = jnp.zeros_like(sum_ref)

    c = conf_ref[...]
    a = accv_ref[...]
    for i in range(_NBINS):
        lo = bounds_ref[0, i]
        hi = bounds_ref[1, i]
        inside = (c > lo) & (c <= hi)
        insf = inside.astype(jnp.float32)
        cnt_ref[i : i + 1, :] += jnp.sum(insf, axis=0, keepdims=True)
        sum_ref[i : i + 1, :] += jnp.sum(jnp.where(inside, a, 0.0), axis=0, keepdims=True)

    @pl.when(step == nsteps - 1)
    def _fin():
        cr = jnp.sum(cnt_ref[...], axis=1, keepdims=True)
        ar = jnp.sum(sum_ref[...], axis=1, keepdims=True)
        out_ref[...] = jnp.where(cr > 0, ar / jnp.maximum(cr, 1.0), 0.0)


@jax.jit
def kernel(logits, targets):
    tgt32 = targets.astype(jnp.int32)
    tgt_tc = tgt32[:_NTC].reshape(_NBLKA, 1, _BLK)

    conf_tc, acc_tc = pl.pallas_call(
        _rows_kernel,
        grid=(_NBLKA,),
        in_specs=[
            pl.BlockSpec((_BLK, _C), lambda i: (i, 0)),
            pl.BlockSpec((1, 1, _BLK), lambda i: (i, 0, 0)),
        ],
        out_specs=[
            pl.BlockSpec((1, 1, _BLK), lambda i: (i, 0, 0)),
            pl.BlockSpec((1, 1, _BLK), lambda i: (i, 0, 0)),
        ],
        out_shape=[
            jax.ShapeDtypeStruct((_NBLKA, 1, _BLK), jnp.float32),
            jax.ShapeDtypeStruct((_NBLKA, 1, _BLK), jnp.float32),
        ],
    )(logits, tgt_tc)

    sc_kernel = functools.partial(
        pl.kernel,
        mesh=plsc.VectorSubcoreMesh(core_axis_name="c", subcore_axis_name="s"),
        out_type=[
            jax.ShapeDtypeStruct((_NSC,), jnp.float32),
            jax.ShapeDtypeStruct((_NSC,), jnp.float32),
        ],
        scratch_types=[
            pltpu.VMEM((_CHUNK * _C + 16,), jnp.float32),
            pltpu.VMEM((_CHUNK,), jnp.int32),
            pltpu.VMEM((_CHUNK,), jnp.float32),
            pltpu.VMEM((_CHUNK,), jnp.float32),
        ],
    )(_sc_rows)
    conf_sc, acc_sc = sc_kernel(logits.reshape(_N * _C), tgt32)

    conf = jnp.concatenate([conf_tc.reshape(_NTC), conf_sc])
    accv = jnp.concatenate([acc_tc.reshape(_NTC), acc_sc])

    interval = jnp.linspace(0.0, 1.0, _NBINS + 1)
    bounds = jnp.zeros((2, _NBINS), jnp.float32)
    bounds = bounds.at[0, :].set(interval[:-1]).at[1, :].set(interval[1:])

    nsteps_b = _BN_ROWS // _BN_BLK
    out = pl.pallas_call(
        functools.partial(_hist_kernel, nsteps=nsteps_b),
        grid=(nsteps_b,),
        in_specs=[
            pl.BlockSpec(memory_space=pltpu.SMEM),
            pl.BlockSpec((_BN_BLK, _BN_COLS), lambda i: (i, 0)),
            pl.BlockSpec((_BN_BLK, _BN_COLS), lambda i: (i, 0)),
        ],
        out_specs=pl.BlockSpec((16, 1), lambda i: (0, 0)),
        out_shape=jax.ShapeDtypeStruct((16, 1), jnp.float32),
        scratch_shapes=[
            pltpu.VMEM((16, _BN_COLS), jnp.float32),
            pltpu.VMEM((16, _BN_COLS), jnp.float32),
        ],
    )(bounds, conf.reshape(_BN_ROWS, _BN_COLS), accv.reshape(_BN_ROWS, _BN_COLS))

    return out[:_NBINS, :]


# speedup vs baseline: 2.2098x; 2.2098x over previous
import functools

import jax
import jax.numpy as jnp
from jax.experimental import pallas as pl
from jax.experimental.pallas import tpu as pltpu

_N = 500000
_C = 100
_NBINS = 10
_BLK = 20000
_NBLKA = _N // _BLK
_BN_ROWS = 5000
_BN_COLS = 100
_BN_BLK = 1000


def _rows_kernel(x_ref, tgt_ref, conf_ref, acc_ref):
    xt = jnp.swapaxes(x_ref[...], 0, 1)
    m = jnp.max(xt, axis=0, keepdims=True)
    e = jnp.exp(xt - m)
    s = jnp.sum(e, axis=0, keepdims=True)
    conf_ref[0] = 1.0 / s
    iota = jax.lax.broadcasted_iota(jnp.int32, xt.shape, 0)
    tval = jnp.max(jnp.where(iota == tgt_ref[0], xt, -jnp.inf), axis=0, keepdims=True)
    acc_ref[0] = (tval == m).astype(jnp.float32)


def _hist_kernel(bounds_ref, conf_ref, accv_ref, out_ref, cnt_ref, sum_ref, *, nsteps):
    step = pl.program_id(0)

    @pl.when(step == 0)
    def _init():
        cnt_ref[...] = jnp.zeros_like(cnt_ref)
        sum_ref[...] = jnp.zeros_like(sum_ref)

    c = conf_ref[...]
    a = accv_ref[...]
    for i in range(_NBINS):
        lo = bounds_ref[0, i]
        hi = bounds_ref[1, i]
        inside = (c > lo) & (c <= hi)
        insf = inside.astype(jnp.float32)
        cnt_ref[i : i + 1, :] += jnp.sum(insf, axis=0, keepdims=True)
        sum_ref[i : i + 1, :] += jnp.sum(jnp.where(inside, a, 0.0), axis=0, keepdims=True)

    @pl.when(step == nsteps - 1)
    def _fin():
        cr = jnp.sum(cnt_ref[...], axis=1, keepdims=True)
        ar = jnp.sum(sum_ref[...], axis=1, keepdims=True)
        out_ref[...] = jnp.where(cr > 0, ar / jnp.maximum(cr, 1.0), 0.0)


@jax.jit
def kernel(logits, targets):
    tgt = targets.astype(jnp.int32).reshape(_NBLKA, 1, _BLK)

    conf, accv = pl.pallas_call(
        _rows_kernel,
        grid=(_NBLKA,),
        in_specs=[
            pl.BlockSpec((_BLK, _C), lambda i: (i, 0)),
            pl.BlockSpec((1, 1, _BLK), lambda i: (i, 0, 0)),
        ],
        out_specs=[
            pl.BlockSpec((1, 1, _BLK), lambda i: (i, 0, 0)),
            pl.BlockSpec((1, 1, _BLK), lambda i: (i, 0, 0)),
        ],
        out_shape=[
            jax.ShapeDtypeStruct((_NBLKA, 1, _BLK), jnp.float32),
            jax.ShapeDtypeStruct((_NBLKA, 1, _BLK), jnp.float32),
        ],
    )(logits, tgt)

    interval = jnp.linspace(0.0, 1.0, _NBINS + 1)
    bounds = jnp.zeros((2, _NBINS), jnp.float32)
    bounds = bounds.at[0, :].set(interval[:-1]).at[1, :].set(interval[1:])

    nsteps_b = _BN_ROWS // _BN_BLK
    out = pl.pallas_call(
        functools.partial(_hist_kernel, nsteps=nsteps_b),
        grid=(nsteps_b,),
        in_specs=[
            pl.BlockSpec(memory_space=pltpu.SMEM),
            pl.BlockSpec((_BN_BLK, _BN_COLS), lambda i: (i, 0)),
            pl.BlockSpec((_BN_BLK, _BN_COLS), lambda i: (i, 0)),
        ],
        out_specs=pl.BlockSpec((16, 1), lambda i: (0, 0)),
        out_shape=jax.ShapeDtypeStruct((16, 1), jnp.float32),
        scratch_shapes=[
            pltpu.VMEM((16, _BN_COLS), jnp.float32),
            pltpu.VMEM((16, _BN_COLS), jnp.float32),
        ],
    )(bounds, conf.reshape(_BN_ROWS, _BN_COLS), accv.reshape(_BN_ROWS, _BN_COLS))

    return out[:_NBINS, :]


# speedup vs baseline: 2.2770x; 1.0304x over previous
import functools

import jax
import jax.numpy as jnp
from jax.experimental import pallas as pl
from jax.experimental.pallas import tpu as pltpu

_N = 500000
_C = 100
_NBINS = 10
_BLK = 25000
_NBLKA = _N // _BLK
_BN_ROWS = 5000
_BN_COLS = 100
_BN_BLK = 1000


def _rows_kernel(x_ref, tgt_ref, pk_ref):
    xt = jnp.swapaxes(x_ref[...], 0, 1)
    m = jnp.max(xt, axis=0, keepdims=True)
    e = jnp.exp(xt - m)
    s = jnp.sum(e, axis=0, keepdims=True)
    conf = 1.0 / s
    iota = jax.lax.broadcasted_iota(jnp.int32, xt.shape, 0)
    tval = jnp.max(jnp.where(iota == tgt_ref[0], xt, -jnp.inf), axis=0, keepdims=True)
    pk_ref[0] = jnp.where(tval == m, conf, -conf)


def _hist_kernel(bounds_ref, conf_ref, out_ref, cnt_ref, sum_ref, *, nsteps):
    step = pl.program_id(0)

    @pl.when(step == 0)
    def _init():
        cnt_ref[...] = jnp.zeros_like(cnt_ref)
        sum_ref[...] = jnp.zeros_like(sum_ref)

    pk = conf_ref[...]
    c = jnp.abs(pk)
    a = jnp.where(pk > 0, 1.0, 0.0)
    for i in range(_NBINS):
        lo = bounds_ref[0, i]
        hi = bounds_ref[1, i]
        inside = (c > lo) & (c <= hi)
        insf = inside.astype(jnp.float32)
        cnt_ref[i : i + 1, :] += jnp.sum(insf, axis=0, keepdims=True)
        sum_ref[i : i + 1, :] += jnp.sum(jnp.where(inside, a, 0.0), axis=0, keepdims=True)

    @pl.when(step == nsteps - 1)
    def _fin():
        cr = jnp.sum(cnt_ref[...], axis=1, keepdims=True)
        ar = jnp.sum(sum_ref[...], axis=1, keepdims=True)
        out_ref[...] = jnp.where(cr > 0, ar / jnp.maximum(cr, 1.0), 0.0)


@jax.jit
def kernel(logits, targets):
    tgt = targets.astype(jnp.int32).reshape(_NBLKA, 1, _BLK)

    packed = pl.pallas_call(
        _rows_kernel,
        grid=(_NBLKA,),
        in_specs=[
            pl.BlockSpec((_BLK, _C), lambda i: (i, 0)),
            pl.BlockSpec((1, 1, _BLK), lambda i: (i, 0, 0)),
        ],
        out_specs=pl.BlockSpec((1, 1, _BLK), lambda i: (i, 0, 0)),
        out_shape=jax.ShapeDtypeStruct((_NBLKA, 1, _BLK), jnp.float32),
    )(logits, tgt)

    interval = jnp.linspace(0.0, 1.0, _NBINS + 1)
    bounds = jnp.zeros((2, _NBINS), jnp.float32)
    bounds = bounds.at[0, :].set(interval[:-1]).at[1, :].set(interval[1:])

    nsteps_b = _BN_ROWS // _BN_BLK
    out = pl.pallas_call(
        functools.partial(_hist_kernel, nsteps=nsteps_b),
        grid=(nsteps_b,),
        in_specs=[
            pl.BlockSpec(memory_space=pltpu.SMEM),
            pl.BlockSpec((_BN_BLK, _BN_COLS), lambda i: (i, 0)),
        ],
        out_specs=pl.BlockSpec((16, 1), lambda i: (0, 0)),
        out_shape=jax.ShapeDtypeStruct((16, 1), jnp.float32),
        scratch_shapes=[
            pltpu.VMEM((16, _BN_COLS), jnp.float32),
            pltpu.VMEM((16, _BN_COLS), jnp.float32),
        ],
    )(bounds, packed.reshape(_BN_ROWS, _BN_COLS))

    return out[:_NBINS, :]
